# Initial kernel scaffold; baseline (speedup 1.0000x reference)
import functools

import jax
import jax.numpy as jnp
from jax import lax
from jax.experimental import pallas as pl
from jax.experimental.pallas import tpu as pltpu
from jax.experimental.pallas import tpu_sc as plsc

EMB_DIM = 16
HIST = 200
IDX_MINOR = 100
ROWS_PER_CHUNK = 8
IDXROWS_PER_CHUNK = ROWS_PER_CHUNK * HIST // IDX_MINOR


def _sc_mean_pool(x2, table, batch):
    info = plsc.get_sparse_core_info()
    nw = info.num_cores * info.num_subcores
    rows_per_w = batch // nw
    chunks = rows_per_w // ROWS_PER_CHUNK
    mesh = plsc.VectorSubcoreMesh(core_axis_name="c", subcore_axis_name="s")

    @functools.partial(
        pl.kernel,
        out_type=jax.ShapeDtypeStruct((batch, EMB_DIM), jnp.float32),
        mesh=mesh,
        scratch_types=[
            pltpu.VMEM((IDXROWS_PER_CHUNK, IDX_MINOR), jnp.int32),
            pltpu.VMEM((IDXROWS_PER_CHUNK, IDX_MINOR, EMB_DIM), jnp.float32),
            pltpu.VMEM((ROWS_PER_CHUNK, EMB_DIM), jnp.float32),
            pltpu.SemaphoreType.DMA,
        ],
    )
    def k(x_hbm, tab_hbm, out_hbm, idx_v, rows_v, means_v, sem):
        wid = lax.axis_index("s") * info.num_cores + lax.axis_index("c")
        idxrow_base = wid * (chunks * IDXROWS_PER_CHUNK)
        brow_base = wid * rows_per_w
        inv = jnp.float32(1.0 / HIST)

        def chunk_body(g, carry):
            pltpu.sync_copy(
                x_hbm.at[pl.ds(idxrow_base + g * IDXROWS_PER_CHUNK,
                               IDXROWS_PER_CHUNK)],
                idx_v)
            descs = [
                pltpu.async_copy(tab_hbm.at[idx_v.at[j]], rows_v.at[j], sem)
                for j in range(IDXROWS_PER_CHUNK)
            ]
            for dsc in descs:
                dsc.wait()
            for r in range(ROWS_PER_CHUNK):
                j0 = 2 * r

                def red(i, acc, j0=j0):
                    a0, a1 = acc
                    kk = i * 4
                    a0 = a0 + rows_v[j0, kk] + rows_v[j0 + 1, kk]
                    a1 = a1 + rows_v[j0, kk + 1] + rows_v[j0 + 1, kk + 1]
                    a0 = a0 + rows_v[j0, kk + 2] + rows_v[j0 + 1, kk + 2]
                    a1 = a1 + rows_v[j0, kk + 3] + rows_v[j0 + 1, kk + 3]
                    return a0, a1

                z = jnp.zeros((EMB_DIM,), jnp.float32)
                a0, a1 = lax.fori_loop(0, IDX_MINOR // 4, red, (z, z))
                means_v[r] = (a0 + a1) * inv
            pltpu.sync_copy(
                means_v,
                out_hbm.at[pl.ds(brow_base + g * ROWS_PER_CHUNK,
                                 ROWS_PER_CHUNK)])
            return carry

        lax.fori_loop(0, chunks, chunk_body, 0)

    return k(x2, table)


def _tc_linear(m, w_t, b):
    batch = m.shape[0]
    blk = 2048

    def body(m_ref, w_ref, b_ref, o_ref):
        o_ref[...] = jnp.dot(m_ref[...], w_ref[...],
                             preferred_element_type=jnp.float32) + b_ref[...]

    return pl.pallas_call(
        body,
        grid=(batch // blk,),
        in_specs=[
            pl.BlockSpec((blk, EMB_DIM), lambda i: (i, 0)),
            pl.BlockSpec((EMB_DIM, 2), lambda i: (0, 0)),
            pl.BlockSpec((1, 2), lambda i: (0, 0)),
        ],
        out_specs=pl.BlockSpec((blk, 2), lambda i: (i, 0)),
        out_shape=jax.ShapeDtypeStruct((batch, 2), jnp.float32),
    )(m, w_t, b.reshape(1, 2))


def kernel(x, emb_table, fc1_w, fc1_b):
    batch, hist = x.shape
    x2 = x.reshape(batch * hist // IDX_MINOR, IDX_MINOR).astype(jnp.int32)
    means = _sc_mean_pool(x2, emb_table, batch)
    return _tc_linear(means, fc1_w.T, fc1_b)


# speedup vs baseline: 7.6941x; 7.6941x over previous
import functools

import jax
import jax.numpy as jnp
from jax import lax
from jax.experimental import pallas as pl
from jax.experimental.pallas import tpu as pltpu
from jax.experimental.pallas import tpu_sc as plsc

EMB_DIM = 16
HIST = 200
IDX_MINOR = 100
ROWS_PER_CHUNK = 8
IDXROWS_PER_CHUNK = ROWS_PER_CHUNK * HIST // IDX_MINOR


def _sc_mean_pool(x2, table, batch):
    info = plsc.get_sparse_core_info()
    nw = info.num_cores * info.num_subcores
    rows_per_w = batch // nw
    chunks = rows_per_w // ROWS_PER_CHUNK
    mesh = plsc.VectorSubcoreMesh(core_axis_name="c", subcore_axis_name="s")

    @functools.partial(
        pl.kernel,
        out_type=jax.ShapeDtypeStruct((batch, EMB_DIM), jnp.float32),
        mesh=mesh,
        scratch_types=[
            pltpu.VMEM((IDXROWS_PER_CHUNK, IDX_MINOR), jnp.int32),
            pltpu.VMEM((IDXROWS_PER_CHUNK, IDX_MINOR, EMB_DIM), jnp.float32),
            pltpu.VMEM((ROWS_PER_CHUNK, EMB_DIM), jnp.float32),
            pltpu.SemaphoreType.DMA,
        ],
        compiler_params=pltpu.CompilerParams(use_tc_tiling_on_sc=False),
    )
    def k(x_hbm, tab_hbm, out_hbm, idx_v, rows_v, means_v, sem):
        wid = lax.axis_index("s") * info.num_cores + lax.axis_index("c")
        idxrow_base = wid * (chunks * IDXROWS_PER_CHUNK)
        brow_base = wid * rows_per_w
        inv = jnp.float32(1.0 / HIST)

        def chunk_body(g, carry):
            pltpu.sync_copy(
                x_hbm.at[pl.ds(idxrow_base + g * IDXROWS_PER_CHUNK,
                               IDXROWS_PER_CHUNK)],
                idx_v)
            descs = [
                pltpu.async_copy(tab_hbm.at[idx_v.at[j]], rows_v.at[j], sem)
                for j in range(IDXROWS_PER_CHUNK)
            ]
            for dsc in descs:
                dsc.wait()
            for r in range(ROWS_PER_CHUNK):
                j0 = 2 * r

                def red(i, acc, j0=j0):
                    a0, a1 = acc
                    kk = i * 4
                    a0 = a0 + rows_v[j0, kk] + rows_v[j0 + 1, kk]
                    a1 = a1 + rows_v[j0, kk + 1] + rows_v[j0 + 1, kk + 1]
                    a0 = a0 + rows_v[j0, kk + 2] + rows_v[j0 + 1, kk + 2]
                    a1 = a1 + rows_v[j0, kk + 3] + rows_v[j0 + 1, kk + 3]
                    return a0, a1

                z = jnp.zeros((EMB_DIM,), jnp.float32)
                a0, a1 = lax.fori_loop(0, IDX_MINOR // 4, red, (z, z))
                means_v[r] = (a0 + a1) * inv
            pltpu.sync_copy(
                means_v,
                out_hbm.at[pl.ds(brow_base + g * ROWS_PER_CHUNK,
                                 ROWS_PER_CHUNK)])
            return carry

        lax.fori_loop(0, chunks, chunk_body, 0)

    return k(x2, table)


def _tc_linear(m, w_t, b):
    batch = m.shape[0]
    blk = 2048

    def body(m_ref, w_ref, b_ref, o_ref):
        o_ref[...] = jnp.dot(m_ref[...], w_ref[...],
                             preferred_element_type=jnp.float32) + b_ref[...]

    return pl.pallas_call(
        body,
        grid=(batch // blk,),
        in_specs=[
            pl.BlockSpec((blk, EMB_DIM), lambda i: (i, 0)),
            pl.BlockSpec((EMB_DIM, 2), lambda i: (0, 0)),
            pl.BlockSpec((1, 2), lambda i: (0, 0)),
        ],
        out_specs=pl.BlockSpec((blk, 2), lambda i: (i, 0)),
        out_shape=jax.ShapeDtypeStruct((batch, 2), jnp.float32),
    )(m, w_t, b.reshape(1, 2))


def kernel(x, emb_table, fc1_w, fc1_b):
    batch, hist = x.shape
    x2 = x.reshape(batch * hist // IDX_MINOR, IDX_MINOR).astype(jnp.int32)
    means = _sc_mean_pool(x2, emb_table, batch)
    return _tc_linear(means, fc1_w.T, fc1_b)
